# baseline (speedup 1.0000x reference)
import functools

import jax
import jax.numpy as jnp
from jax import lax
from jax.experimental import pallas as pl
from jax.experimental.pallas import tpu as pltpu
from jax.experimental.pallas import tpu_sc as plsc

N, C, H, W = 8, 19, 384, 384
L = H * W
NW = 32
SPAN = L // NW
P = 1536
CHUNKS_PER_N = SPAN // P
T = N * CHUNKS_PER_N
UNROLL = 2
CPAD = 32
SHIFT = 18
MASK18 = (1 << SHIFT) - 1


def _tree_argmax(vals):
    def idxvec(node):
        _, i, c = node
        return i if i is not None else jnp.full((16,), c, jnp.int32)

    nodes = [(vals[c], None, c) for c in range(C)]
    while len(nodes) > 1:
        nxt = []
        for j in range(0, len(nodes) - 1, 2):
            va, _, _ = nodes[j]
            vb, _, _ = nodes[j + 1]
            m = vb > va
            v = jnp.maximum(va, vb)
            i = jnp.where(m, idxvec(nodes[j + 1]), idxvec(nodes[j]))
            nxt.append((v, i, None))
        if len(nodes) % 2:
            nxt.append(nodes[-1])
        nodes = nxt
    return idxvec(nodes[0])


@functools.partial(
    pl.kernel,
    out_type=jax.ShapeDtypeStruct((NW, 2, N, CPAD), jnp.float32),
    mesh=plsc.VectorSubcoreMesh(core_axis_name="c", subcore_axis_name="s"),
    compiler_params=pltpu.CompilerParams(needs_layout_passes=False),
    scratch_types=[
        pltpu.VMEM((2, C, P), jnp.float32),
        pltpu.VMEM((2, P), jnp.int32),
        pltpu.VMEM((16 * N * CPAD,), jnp.int32),
        pltpu.VMEM((2, N, CPAD), jnp.float32),
        pltpu.SemaphoreType.DMA,
        pltpu.SemaphoreType.DMA,
        pltpu.SemaphoreType.DMA,
        pltpu.SemaphoreType.DMA,
    ],
)
def _sc_counts(inp_hbm, tgt_hbm, out_hbm, ibuf, tbuf, bins, outv,
               sem_i0, sem_i1, sem_t0, sem_t1):
    sem_i = (sem_i0, sem_i1)
    sem_t = (sem_t0, sem_t1)
    wid = lax.axis_index("s") * 2 + lax.axis_index("c")
    base = wid * SPAN

    def _zero(i, carry):
        bins[pl.ds(i * 16, 16)] = jnp.zeros((16,), jnp.int32)
        return carry

    lax.fori_loop(0, (16 * N * CPAD) // 16, _zero, 0)

    def issue(t, b):
        n = t // CHUNKS_PER_N
        off = base + (t % CHUNKS_PER_N) * P
        pltpu.async_copy(inp_hbm.at[n, :, pl.ds(off, P)], ibuf.at[b], sem_i[b])
        pltpu.async_copy(tgt_hbm.at[n, pl.ds(off, P)], tbuf.at[b], sem_t[b])

    def wait(b):
        pltpu.make_async_copy(
            inp_hbm.at[0, :, pl.ds(0, P)], ibuf.at[b], sem_i[b]).wait()
        pltpu.make_async_copy(
            tgt_hbm.at[0, pl.ds(0, P)], tbuf.at[b], sem_t[b]).wait()

    lanes = lax.iota(jnp.int32, 16)
    one = jnp.full((16,), 1, jnp.int32)
    one_tp = jnp.full((16,), 1 + (1 << SHIFT), jnp.int32)

    def compute(t, b):
        nbase = (t // CHUNKS_PER_N) * CPAD

        def body(i, carry):
            for u in range(UNROLL):
                sl = pl.ds(i * (16 * UNROLL) + u * 16, 16)
                pred = _tree_argmax([ibuf[b, c, sl] for c in range(C)])
                tgtv = tbuf[b, sl]
                val = jnp.where(pred == tgtv, one_tp, one)
                idx = lanes * (N * CPAD) + (nbase + tgtv)
                plsc.addupdate_scatter(bins, [idx], val)
            return carry

        lax.fori_loop(0, P // (16 * UNROLL), body, 0)

    issue(0, 0)

    def outer(t0, carry):
        for b in range(2):
            t = t0 * 2 + b

            @pl.when(t + 1 < T)
            def _():
                issue(t + 1, (b + 1) % 2)

            wait(b)
            compute(t, b)
        return carry

    lax.fori_loop(0, T // 2, outer, 0)

    for n in range(N):
        for cg in range(2):
            bofs = n * CPAD + cg * 16
            s = bins[pl.ds(bofs, 16)]
            for lane in range(1, 16):
                s = s + bins[pl.ds(lane * (N * CPAD) + bofs, 16)]
            outv[0, n, pl.ds(cg * 16, 16)] = (
                lax.shift_right_logical(s, SHIFT).astype(jnp.float32))
            outv[1, n, pl.ds(cg * 16, 16)] = (s & MASK18).astype(jnp.float32)
    pltpu.sync_copy(outv, out_hbm.at[wid])


def _finalize_kernel(p_ref, o_ref):
    p = p_ref[...]
    tp = jnp.sum(p[:, 0, :, :], axis=0)
    tt = jnp.sum(p[:, 1, :, :], axis=0)
    cidx = lax.broadcasted_iota(jnp.int32, (N, CPAD), 1)
    recall = jnp.where(cidx < C, (tp + 1e-5) / (tt + 1e-5), 0.0)
    o_ref[...] = jnp.broadcast_to(1.0 - jnp.sum(recall) / (N * C), (1, 1))


def kernel(input, target):
    inp = input.reshape(N, C, L)
    tgt = target.reshape(N, L)
    partials = _sc_counts(inp, tgt)
    loss = pl.pallas_call(
        _finalize_kernel,
        out_shape=jax.ShapeDtypeStruct((1, 1), jnp.float32),
    )(partials)
    return loss[0, 0]
